# Initial kernel scaffold; baseline (speedup 1.0000x reference)
import jax
import jax.numpy as jnp
from jax.experimental import pallas as pl


def kernel(edge_indices, coefficients, col_features, row_features, params):
    raise NotImplementedError("write your pallas kernel here")


# speedup vs baseline: 2.6626x; 2.6626x over previous
import functools

import jax
import jax.numpy as jnp
from jax import lax
from jax.experimental import pallas as pl
from jax.experimental.pallas import tpu as pltpu
from jax.experimental.pallas import tpu_sc as plsc

EMBD = 32


def _dot(x, w):
    return lax.dot_general(x, w, (((1,), (1,)), ((), ())),
                           preferred_element_type=jnp.float32)


def _ln_last(x, g, b, eps=1e-5):
    m = jnp.mean(x, axis=-1, keepdims=True)
    v = jnp.mean((x - m) ** 2, axis=-1, keepdims=True)
    return (x - m) / jnp.sqrt(v + eps) * g + b


def _embed_proj_body(rf, cf,
                     re_g, re_b, re_w1, re_b1, re_w2, re_b2,
                     ce_g, ce_b, ce_w1, ce_b1, ce_w2, ce_b2,
                     fl_w, fl_bias, fr_w,
                     remb_o, cemb_o, a1_o, b1_o):
    r = _ln_last(rf[...], re_g[...], re_b[...])
    r = jnp.maximum(_dot(r, re_w1[...]) + re_b1[...], 0.0)
    r = jnp.maximum(_dot(r, re_w2[...]) + re_b2[...], 0.0)
    c = _ln_last(cf[...], ce_g[...], ce_b[...])
    c = jnp.maximum(_dot(c, ce_w1[...]) + ce_b1[...], 0.0)
    c = jnp.maximum(_dot(c, ce_w2[...]) + ce_b2[...], 0.0)
    remb_o[...] = r
    cemb_o[...] = c
    a1_o[...] = _dot(r, fl_w[...]) + fl_bias[...]
    b1_o[...] = _dot(c, fr_w[...])


def _edge_ln_body(s_in, g, bb, fw, fb, o):
    t = jnp.maximum(_ln_last(s_in[...], g[...], bb[...]), 0.0)
    o[...] = _dot(t, fw[...]) + fb[...]


def _post_body(p, remb, cemb,
               pc_g, pc_b, o1a, o1b, o1bias, o2w, o2b,
               fl2_w, fl2_bias, fr2_w,
               a2_o, b2_o):
    z = p[0] + p[1]
    h = _ln_last(z, pc_g[...], pc_b[...])
    u = jnp.maximum(_dot(h, o1a[...]) + _dot(remb[...], o1b[...]) + o1bias[...], 0.0)
    row2 = _dot(u, o2w[...]) + o2b[...]
    a2_o[...] = _dot(cemb[...], fl2_w[...]) + fl2_bias[...]
    b2_o[...] = _dot(row2, fr2_w[...])


def _head_body(p, cemb,
               pc_g, pc_b, o1a, o1b, o1bias, o2w, o2b,
               w1, b1, w2, inv_n, out_o):
    z = p[0] + p[1]
    h = _ln_last(z, pc_g[...], pc_b[...])
    u = jnp.maximum(_dot(h, o1a[...]) + _dot(cemb[...], o1b[...]) + o1bias[...], 0.0)
    col2 = _dot(u, o2w[...]) + o2b[...]
    y = jnp.maximum(_dot(col2, w1[...]) + b1[...], 0.0)
    y = jnp.maximum(_dot(y, w2[...]), 0.0)
    nrm = jnp.maximum(jnp.sum(jnp.abs(y), axis=-1, keepdims=True), 1e-12)
    part = jnp.sum(y / nrm, axis=0, keepdims=True) * inv_n[0, 0]

    @pl.when(pl.program_id(0) == 0)
    def _():
        out_o[...] = jnp.zeros_like(out_o)

    out_o[...] += part


_K = 80


def _sc_gather(a_tab, b_tab, dst, src):
    e = dst.shape[0]
    info = plsc.get_sparse_core_info()
    nw = info.num_cores * info.num_subcores
    per_w = e // nw
    nblk = per_w // _K
    assert per_w % _K == 0 and per_w % 8 == 0

    mesh = plsc.VectorSubcoreMesh(core_axis_name="c", subcore_axis_name="s")

    @functools.partial(
        pl.kernel, mesh=mesh,
        out_type=jax.ShapeDtypeStruct((e, EMBD), jnp.float32),
        compiler_params=pltpu.CompilerParams(use_tc_tiling_on_sc=False),
        scratch_types=[
            pltpu.VMEM((_K,), jnp.int32),
            pltpu.VMEM((_K,), jnp.int32),
            pltpu.VMEM((_K, EMBD), jnp.float32),
            pltpu.VMEM((_K, EMBD), jnp.float32),
            pltpu.SemaphoreType.DMA,
            pltpu.SemaphoreType.DMA,
        ])
    def k(a_hbm, b_hbm, dst_hbm, src_hbm, out_hbm,
          idx_d, idx_s, rows_a, rows_b, sem_a, sem_b):
        cid = lax.axis_index("c")
        sid = lax.axis_index("s")
        wid = sid * info.num_cores + cid

        def body(i, carry):
            base = wid * per_w + i * _K
            pltpu.sync_copy(dst_hbm.at[pl.ds(base, _K)], idx_d)
            pltpu.sync_copy(src_hbm.at[pl.ds(base, _K)], idx_s)
            ca = pltpu.async_copy(a_hbm.at[idx_d], rows_a, sem_a)
            cb = pltpu.async_copy(b_hbm.at[idx_s], rows_b, sem_b)
            ca.wait()
            cb.wait()
            for j in range(_K * EMBD // 16):
                r, c = j // (EMBD // 16), (j % (EMBD // 16)) * 16
                rows_a[r, pl.ds(c, 16)] = (rows_a[r, pl.ds(c, 16)] +
                                           rows_b[r, pl.ds(c, 16)])
            pltpu.sync_copy(rows_a, out_hbm.at[pl.ds(base, _K)])
            return carry

        lax.fori_loop(0, nblk, body, 0)

    return k(a_tab, b_tab, dst, src)


def _sc_scatter(t, dst, zeros, n):
    e = dst.shape[0]
    w = t.shape[1]
    info = plsc.get_sparse_core_info()
    nc, ns = info.num_cores, info.num_subcores
    nw = nc * ns
    per_w = e // nw
    nblk = per_w // _K
    assert per_w % _K == 0
    base_rows = (n // ns) // 8 * 8
    last_rows = n - base_rows * (ns - 1)

    mesh = plsc.VectorSubcoreMesh(core_axis_name="c", subcore_axis_name="s")

    @functools.partial(
        pl.kernel, mesh=mesh,
        out_type=jax.ShapeDtypeStruct((nc, n, w), jnp.float32),
        compiler_params=pltpu.CompilerParams(use_tc_tiling_on_sc=False),
        scratch_types=[
            pltpu.VMEM((_K,), jnp.int32),
            pltpu.VMEM((_K, w), jnp.float32),
            pltpu.VMEM_SHARED((n, w), jnp.float32),
        ])
    def k(t_hbm, dst_hbm, z_hbm, out_hbm, idx_v, rows_v, acc):
        cid = lax.axis_index("c")
        sid = lax.axis_index("s")
        wid = sid * nc + cid

        @pl.when(sid == 0)
        def _():
            pltpu.sync_copy(z_hbm, acc)

        plsc.subcore_barrier()

        def body(i, carry):
            base = wid * per_w + i * _K
            pltpu.sync_copy(dst_hbm.at[pl.ds(base, _K)], idx_v)
            pltpu.sync_copy(t_hbm.at[pl.ds(base, _K)], rows_v)
            pltpu.sync_copy(rows_v, acc.at[idx_v], add=True)
            return carry

        lax.fori_loop(0, nblk, body, 0)
        plsc.subcore_barrier()

        @pl.when(sid < ns - 1)
        def _():
            pltpu.sync_copy(acc.at[pl.ds(sid * base_rows, base_rows)],
                            out_hbm.at[cid, pl.ds(sid * base_rows, base_rows)])

        @pl.when(sid == ns - 1)
        def _():
            pltpu.sync_copy(acc.at[pl.ds((ns - 1) * base_rows, last_rows)],
                            out_hbm.at[cid, pl.ds((ns - 1) * base_rows, last_rows)])

    return k(t, dst, zeros)


def _full(shape):
    return pl.BlockSpec(shape, lambda i: tuple(0 for _ in shape))


def _v2d(x):
    return x.reshape(1, -1)


def kernel(edge_indices, coefficients, col_features, row_features, params):
    del coefficients
    p = params
    n = row_features.shape[0]
    e = edge_indices.shape[1]
    src = edge_indices[0]
    dst = edge_indices[1]

    cef = p['ee_ln_b'][0] * p['c2r_fe_W'][:, 0]
    cef2 = p['ee_ln_b'][0] * p['r2c_fe_W'][:, 0]

    bn = 5000
    grid_n = n // bn

    emb_args = (row_features, col_features,
                _v2d(p['re_ln_g']), _v2d(p['re_ln_b']), p['re_W1'],
                _v2d(p['re_b1']), p['re_W2'], _v2d(p['re_b2']),
                _v2d(p['ce_ln_g']), _v2d(p['ce_ln_b']), p['ce_W1'],
                _v2d(p['ce_b1']), p['ce_W2'], _v2d(p['ce_b2']),
                p['c2r_fl_W'], _v2d(p['c2r_fl_b'] + cef), p['c2r_fr_W'])
    in_specs = [pl.BlockSpec((bn, 7), lambda i: (i, 0)),
                pl.BlockSpec((bn, 7), lambda i: (i, 0))]
    in_specs += [_full(a.shape) for a in emb_args[2:]]
    remb, cemb, a1, b1 = pl.pallas_call(
        _embed_proj_body,
        grid=(grid_n,),
        in_specs=in_specs,
        out_specs=[pl.BlockSpec((bn, EMBD), lambda i: (i, 0))] * 4,
        out_shape=[jax.ShapeDtypeStruct((n, EMBD), jnp.float32)] * 4,
    )(*emb_args)

    zeros = jnp.zeros((n, EMBD), jnp.float32)

    def conv(a_tab, b_tab, sidx, didx, pre):
        s = _sc_gather(a_tab, b_tab, didx, sidx)
        bk = 4000
        t = pl.pallas_call(
            _edge_ln_body,
            grid=(e // bk,),
            in_specs=[pl.BlockSpec((bk, EMBD), lambda i: (i, 0)),
                      _full((1, EMBD)), _full((1, EMBD)),
                      _full((EMBD, EMBD)), _full((1, EMBD))],
            out_specs=pl.BlockSpec((bk, EMBD), lambda i: (i, 0)),
            out_shape=jax.ShapeDtypeStruct((e, EMBD), jnp.float32),
        )(s, _v2d(p[pre + 'ffin_ln_g']), _v2d(p[pre + 'ffin_ln_b']),
          p[pre + 'ffin_W'], _v2d(p[pre + 'ffin_b']))
        return _sc_scatter(t, didx, zeros, n)

    parts1 = conv(a1, b1, src, dst, 'c2r_')

    post_args = (parts1, remb, cemb,
                 _v2d(p['c2r_pc_g']), _v2d(p['c2r_pc_b']),
                 p['c2r_o1_W'][:, :EMBD], p['c2r_o1_W'][:, EMBD:],
                 _v2d(p['c2r_o1_b']), p['c2r_o2_W'], _v2d(p['c2r_o2_b']),
                 p['r2c_fl_W'], _v2d(p['r2c_fl_b'] + cef2), p['r2c_fr_W'])
    in_specs = [pl.BlockSpec((2, bn, EMBD), lambda i: (0, i, 0)),
                pl.BlockSpec((bn, EMBD), lambda i: (i, 0)),
                pl.BlockSpec((bn, EMBD), lambda i: (i, 0))]
    in_specs += [_full(a.shape) for a in post_args[3:]]
    a2, b2 = pl.pallas_call(
        _post_body,
        grid=(grid_n,),
        in_specs=in_specs,
        out_specs=[pl.BlockSpec((bn, EMBD), lambda i: (i, 0))] * 2,
        out_shape=[jax.ShapeDtypeStruct((n, EMBD), jnp.float32)] * 2,
    )(*post_args)

    parts2 = conv(a2, b2, dst, src, 'r2c_')

    head_args = (parts2, cemb,
                 _v2d(p['r2c_pc_g']), _v2d(p['r2c_pc_b']),
                 p['r2c_o1_W'][:, :EMBD], p['r2c_o1_W'][:, EMBD:],
                 _v2d(p['r2c_o1_b']), p['r2c_o2_W'], _v2d(p['r2c_o2_b']),
                 p['out_W1'], _v2d(p['out_b1']), p['out_W2'],
                 jnp.full((1, 1), 1.0 / n, jnp.float32))
    in_specs = [pl.BlockSpec((2, bn, EMBD), lambda i: (0, i, 0)),
                pl.BlockSpec((bn, EMBD), lambda i: (i, 0))]
    in_specs += [_full(a.shape) for a in head_args[2:]]
    out = pl.pallas_call(
        _head_body,
        grid=(grid_n,),
        in_specs=in_specs,
        out_specs=pl.BlockSpec((1, 4), lambda i: (0, 0)),
        out_shape=jax.ShapeDtypeStruct((1, 4), jnp.float32),
    )(*head_args)

    return out.reshape(4)


# speedup vs baseline: 3.1901x; 1.1981x over previous
import functools

import jax
import jax.numpy as jnp
from jax import lax
from jax.experimental import pallas as pl
from jax.experimental.pallas import tpu as pltpu
from jax.experimental.pallas import tpu_sc as plsc

EMBD = 32


def _dot(x, w):
    return lax.dot_general(x, w, (((1,), (1,)), ((), ())),
                           preferred_element_type=jnp.float32)


def _ln_last(x, g, b, eps=1e-5):
    m = jnp.mean(x, axis=-1, keepdims=True)
    v = jnp.mean((x - m) ** 2, axis=-1, keepdims=True)
    return (x - m) / jnp.sqrt(v + eps) * g + b


def _embed_proj_body(rf, cf,
                     re_g, re_b, re_w1, re_b1, re_w2, re_b2,
                     ce_g, ce_b, ce_w1, ce_b1, ce_w2, ce_b2,
                     fl_w, fl_bias, fr_w,
                     remb_o, cemb_o, a1_o, b1_o):
    r = _ln_last(rf[...], re_g[...], re_b[...])
    r = jnp.maximum(_dot(r, re_w1[...]) + re_b1[...], 0.0)
    r = jnp.maximum(_dot(r, re_w2[...]) + re_b2[...], 0.0)
    c = _ln_last(cf[...], ce_g[...], ce_b[...])
    c = jnp.maximum(_dot(c, ce_w1[...]) + ce_b1[...], 0.0)
    c = jnp.maximum(_dot(c, ce_w2[...]) + ce_b2[...], 0.0)
    remb_o[...] = r
    cemb_o[...] = c
    a1_o[...] = _dot(r, fl_w[...]) + fl_bias[...]
    b1_o[...] = _dot(c, fr_w[...])


def _edge_ln_body(s_in, g, bb, fw, fb, o):
    t = jnp.maximum(_ln_last(s_in[...], g[...], bb[...]), 0.0)
    o[...] = _dot(t, fw[...]) + fb[...]


def _post_body(p, remb, cemb,
               pc_g, pc_b, o1a, o1b, o1bias, o2w, o2b,
               fl2_w, fl2_bias, fr2_w,
               a2_o, b2_o):
    z = p[0] + p[1]
    h = _ln_last(z, pc_g[...], pc_b[...])
    u = jnp.maximum(_dot(h, o1a[...]) + _dot(remb[...], o1b[...]) + o1bias[...], 0.0)
    row2 = _dot(u, o2w[...]) + o2b[...]
    a2_o[...] = _dot(cemb[...], fl2_w[...]) + fl2_bias[...]
    b2_o[...] = _dot(row2, fr2_w[...])


def _head_body(p, cemb,
               pc_g, pc_b, o1a, o1b, o1bias, o2w, o2b,
               w1, b1, w2, inv_n, out_o):
    z = p[0] + p[1]
    h = _ln_last(z, pc_g[...], pc_b[...])
    u = jnp.maximum(_dot(h, o1a[...]) + _dot(cemb[...], o1b[...]) + o1bias[...], 0.0)
    col2 = _dot(u, o2w[...]) + o2b[...]
    y = jnp.maximum(_dot(col2, w1[...]) + b1[...], 0.0)
    y = jnp.maximum(_dot(y, w2[...]), 0.0)
    nrm = jnp.maximum(jnp.sum(jnp.abs(y), axis=-1, keepdims=True), 1e-12)
    part = jnp.sum(y / nrm, axis=0, keepdims=True) * inv_n[0, 0]

    @pl.when(pl.program_id(0) == 0)
    def _():
        out_o[...] = jnp.zeros_like(out_o)

    out_o[...] += part


_K = 80
_NBUF = 5


def _sc_gather(a_tab, b_tab, dst, src):
    e = dst.shape[0]
    info = plsc.get_sparse_core_info()
    nw = info.num_cores * info.num_subcores
    per_w = e // nw
    nblk = per_w // _K
    assert per_w % _K == 0 and per_w % 8 == 0

    mesh = plsc.VectorSubcoreMesh(core_axis_name="c", subcore_axis_name="s")

    nbuf = _NBUF
    assert nblk % nbuf == 0

    @functools.partial(
        pl.kernel, mesh=mesh,
        out_type=jax.ShapeDtypeStruct((e, EMBD), jnp.float32),
        compiler_params=pltpu.CompilerParams(use_tc_tiling_on_sc=False),
        scratch_types=(
            [pltpu.VMEM((_K,), jnp.int32)] * nbuf +
            [pltpu.VMEM((_K,), jnp.int32)] * nbuf +
            [pltpu.VMEM((_K, EMBD), jnp.float32)] * nbuf +
            [pltpu.VMEM((_K, EMBD), jnp.float32)] * nbuf +
            [pltpu.SemaphoreType.DMA] * nbuf +
            [pltpu.SemaphoreType.DMA] * nbuf))
    def k(a_hbm, b_hbm, dst_hbm, src_hbm, out_hbm, *scr):
        idx_d = scr[0:nbuf]
        idx_s = scr[nbuf:2 * nbuf]
        rows_a = scr[2 * nbuf:3 * nbuf]
        rows_b = scr[3 * nbuf:4 * nbuf]
        sem_a = scr[4 * nbuf:5 * nbuf]
        sem_b = scr[5 * nbuf:6 * nbuf]
        cid = lax.axis_index("c")
        sid = lax.axis_index("s")
        wid = sid * info.num_cores + cid

        def fire(blk, b):
            base = wid * per_w + blk * _K
            pltpu.sync_copy(dst_hbm.at[pl.ds(base, _K)], idx_d[b])
            pltpu.sync_copy(src_hbm.at[pl.ds(base, _K)], idx_s[b])
            pltpu.async_copy(a_hbm.at[idx_d[b]], rows_a[b], sem_a[b])
            pltpu.async_copy(b_hbm.at[idx_s[b]], rows_b[b], sem_b[b])

        for b in range(nbuf):
            fire(b, b)

        def outer(g, carry):
            for b in range(nbuf):
                blk = g * nbuf + b
                base = wid * per_w + blk * _K
                pltpu.make_async_copy(a_hbm.at[idx_d[b]], rows_a[b],
                                      sem_a[b]).wait()
                pltpu.make_async_copy(b_hbm.at[idx_s[b]], rows_b[b],
                                      sem_b[b]).wait()
                for j in range(_K * EMBD // 16):
                    r, c = j // (EMBD // 16), (j % (EMBD // 16)) * 16
                    rows_a[b][r, pl.ds(c, 16)] = (rows_a[b][r, pl.ds(c, 16)] +
                                                  rows_b[b][r, pl.ds(c, 16)])
                pltpu.sync_copy(rows_a[b], out_hbm.at[pl.ds(base, _K)])

                @pl.when(blk + nbuf < nblk)
                def _():
                    fire(blk + nbuf, b)
            return carry

        lax.fori_loop(0, nblk // nbuf, outer, 0)

    return k(a_tab, b_tab, dst, src)


def _sc_scatter(t, dst, zeros, n):
    e = dst.shape[0]
    w = t.shape[1]
    info = plsc.get_sparse_core_info()
    nc, ns = info.num_cores, info.num_subcores
    nw = nc * ns
    per_w = e // nw
    nblk = per_w // _K
    assert per_w % _K == 0
    base_rows = (n // ns) // 8 * 8
    last_rows = n - base_rows * (ns - 1)

    mesh = plsc.VectorSubcoreMesh(core_axis_name="c", subcore_axis_name="s")

    @functools.partial(
        pl.kernel, mesh=mesh,
        out_type=jax.ShapeDtypeStruct((nc, n, w), jnp.float32),
        compiler_params=pltpu.CompilerParams(use_tc_tiling_on_sc=False),
        scratch_types=(
            [pltpu.VMEM((_K,), jnp.int32)] * _NBUF +
            [pltpu.VMEM((_K, w), jnp.float32)] * _NBUF +
            [pltpu.SemaphoreType.DMA] * _NBUF +
            [pltpu.SemaphoreType.DMA] * _NBUF +
            [pltpu.VMEM_SHARED((n, w), jnp.float32)]))
    def k(t_hbm, dst_hbm, z_hbm, out_hbm, *scr):
        idx_v = scr[0:_NBUF]
        rows_v = scr[_NBUF:2 * _NBUF]
        sem_l = scr[2 * _NBUF:3 * _NBUF]
        sem_s = scr[3 * _NBUF:4 * _NBUF]
        acc = scr[4 * _NBUF]
        cid = lax.axis_index("c")
        sid = lax.axis_index("s")
        wid = sid * nc + cid

        @pl.when(sid == 0)
        def _():
            pltpu.sync_copy(z_hbm, acc)

        plsc.subcore_barrier()

        def fire(blk, b):
            base = wid * per_w + blk * _K
            pltpu.sync_copy(dst_hbm.at[pl.ds(base, _K)], idx_v[b])
            pltpu.async_copy(t_hbm.at[pl.ds(base, _K)], rows_v[b], sem_l[b])

        for b in range(_NBUF):
            fire(b, b)

        def outer(g, carry):
            for b in range(_NBUF):
                blk = g * _NBUF + b
                base = wid * per_w + blk * _K
                pltpu.make_async_copy(t_hbm.at[pl.ds(base, _K)], rows_v[b],
                                      sem_l[b]).wait()
                pltpu.async_copy(rows_v[b], acc.at[idx_v[b]], sem_s[b],
                                 add=True)

                @pl.when(blk + _NBUF < nblk)
                def _():
                    pltpu.make_async_copy(rows_v[b], acc.at[idx_v[b]],
                                          sem_s[b]).wait()
                    fire(blk + _NBUF, b)
            return carry

        lax.fori_loop(0, nblk // _NBUF, outer, 0)
        for b in range(_NBUF):
            pltpu.make_async_copy(rows_v[b], acc.at[idx_v[b]], sem_s[b]).wait()
        plsc.subcore_barrier()

        @pl.when(sid < ns - 1)
        def _():
            pltpu.sync_copy(acc.at[pl.ds(sid * base_rows, base_rows)],
                            out_hbm.at[cid, pl.ds(sid * base_rows, base_rows)])

        @pl.when(sid == ns - 1)
        def _():
            pltpu.sync_copy(acc.at[pl.ds((ns - 1) * base_rows, last_rows)],
                            out_hbm.at[cid, pl.ds((ns - 1) * base_rows, last_rows)])

    return k(t, dst, zeros)


def _full(shape):
    return pl.BlockSpec(shape, lambda i: tuple(0 for _ in shape))


def _v2d(x):
    return x.reshape(1, -1)


def kernel(edge_indices, coefficients, col_features, row_features, params):
    del coefficients
    p = params
    n = row_features.shape[0]
    e = edge_indices.shape[1]
    src = edge_indices[0]
    dst = edge_indices[1]

    cef = p['ee_ln_b'][0] * p['c2r_fe_W'][:, 0]
    cef2 = p['ee_ln_b'][0] * p['r2c_fe_W'][:, 0]

    bn = 5000
    grid_n = n // bn

    emb_args = (row_features, col_features,
                _v2d(p['re_ln_g']), _v2d(p['re_ln_b']), p['re_W1'],
                _v2d(p['re_b1']), p['re_W2'], _v2d(p['re_b2']),
                _v2d(p['ce_ln_g']), _v2d(p['ce_ln_b']), p['ce_W1'],
                _v2d(p['ce_b1']), p['ce_W2'], _v2d(p['ce_b2']),
                p['c2r_fl_W'], _v2d(p['c2r_fl_b'] + cef), p['c2r_fr_W'])
    in_specs = [pl.BlockSpec((bn, 7), lambda i: (i, 0)),
                pl.BlockSpec((bn, 7), lambda i: (i, 0))]
    in_specs += [_full(a.shape) for a in emb_args[2:]]
    remb, cemb, a1, b1 = pl.pallas_call(
        _embed_proj_body,
        grid=(grid_n,),
        in_specs=in_specs,
        out_specs=[pl.BlockSpec((bn, EMBD), lambda i: (i, 0))] * 4,
        out_shape=[jax.ShapeDtypeStruct((n, EMBD), jnp.float32)] * 4,
    )(*emb_args)

    zeros = jnp.zeros((n, EMBD), jnp.float32)

    def conv(a_tab, b_tab, sidx, didx, pre):
        s = _sc_gather(a_tab, b_tab, didx, sidx)
        bk = 4000
        t = pl.pallas_call(
            _edge_ln_body,
            grid=(e // bk,),
            in_specs=[pl.BlockSpec((bk, EMBD), lambda i: (i, 0)),
                      _full((1, EMBD)), _full((1, EMBD)),
                      _full((EMBD, EMBD)), _full((1, EMBD))],
            out_specs=pl.BlockSpec((bk, EMBD), lambda i: (i, 0)),
            out_shape=jax.ShapeDtypeStruct((e, EMBD), jnp.float32),
        )(s, _v2d(p[pre + 'ffin_ln_g']), _v2d(p[pre + 'ffin_ln_b']),
          p[pre + 'ffin_W'], _v2d(p[pre + 'ffin_b']))
        return _sc_scatter(t, didx, zeros, n)

    parts1 = conv(a1, b1, src, dst, 'c2r_')

    post_args = (parts1, remb, cemb,
                 _v2d(p['c2r_pc_g']), _v2d(p['c2r_pc_b']),
                 p['c2r_o1_W'][:, :EMBD], p['c2r_o1_W'][:, EMBD:],
                 _v2d(p['c2r_o1_b']), p['c2r_o2_W'], _v2d(p['c2r_o2_b']),
                 p['r2c_fl_W'], _v2d(p['r2c_fl_b'] + cef2), p['r2c_fr_W'])
    in_specs = [pl.BlockSpec((2, bn, EMBD), lambda i: (0, i, 0)),
                pl.BlockSpec((bn, EMBD), lambda i: (i, 0)),
                pl.BlockSpec((bn, EMBD), lambda i: (i, 0))]
    in_specs += [_full(a.shape) for a in post_args[3:]]
    a2, b2 = pl.pallas_call(
        _post_body,
        grid=(grid_n,),
        in_specs=in_specs,
        out_specs=[pl.BlockSpec((bn, EMBD), lambda i: (i, 0))] * 2,
        out_shape=[jax.ShapeDtypeStruct((n, EMBD), jnp.float32)] * 2,
    )(*post_args)

    parts2 = conv(a2, b2, dst, src, 'r2c_')

    head_args = (parts2, cemb,
                 _v2d(p['r2c_pc_g']), _v2d(p['r2c_pc_b']),
                 p['r2c_o1_W'][:, :EMBD], p['r2c_o1_W'][:, EMBD:],
                 _v2d(p['r2c_o1_b']), p['r2c_o2_W'], _v2d(p['r2c_o2_b']),
                 p['out_W1'], _v2d(p['out_b1']), p['out_W2'],
                 jnp.full((1, 1), 1.0 / n, jnp.float32))
    in_specs = [pl.BlockSpec((2, bn, EMBD), lambda i: (0, i, 0)),
                pl.BlockSpec((bn, EMBD), lambda i: (i, 0))]
    in_specs += [_full(a.shape) for a in head_args[2:]]
    out = pl.pallas_call(
        _head_body,
        grid=(grid_n,),
        in_specs=in_specs,
        out_specs=pl.BlockSpec((1, 4), lambda i: (0, 0)),
        out_shape=jax.ShapeDtypeStruct((1, 4), jnp.float32),
    )(*head_args)

    return out.reshape(4)
